# baseline (speedup 1.0000x reference)
import functools

import jax
import jax.numpy as jnp
from jax import lax
from jax.experimental import pallas as pl
from jax.experimental.pallas import tpu as pltpu
from jax.experimental.pallas import tpu_sc as plsc

_F = 32
_L = 16


def _mf_body(user_hbm, item_hbm, uft_hbm, ift_hbm, out_hbm,
             uidx_v, iidx_v, uvals_v, ivals_v, out_v,
             usem, isem, *, b_per_w, num_cores):
    wid = lax.axis_index("s") * num_cores + lax.axis_index("c")
    base = wid * b_per_w

    pltpu.sync_copy(user_hbm.at[pl.ds(base, b_per_w)], uidx_v)
    pltpu.sync_copy(item_hbm.at[pl.ds(base, b_per_w)], iidx_v)

    copies = []
    for f in range(_F):
        dst = uvals_v.at[pl.ds(f * b_per_w, b_per_w)]
        copies.append(pltpu.async_copy(uft_hbm.at[f].at[uidx_v], dst, usem))
        dst = ivals_v.at[pl.ds(f * b_per_w, b_per_w)]
        copies.append(pltpu.async_copy(ift_hbm.at[f].at[iidx_v], dst, isem))
    for cp in copies:
        cp.wait()

    n_chunks = b_per_w // _L

    def reduce_chunk(c, carry):
        acc = jnp.zeros((_L,), jnp.float32)
        for f in range(_F):
            u = uvals_v[pl.ds(f * b_per_w + c * _L, _L)]
            v = ivals_v[pl.ds(f * b_per_w + c * _L, _L)]
            acc = acc + u * v
        out_v[pl.ds(c * _L, _L)] = acc
        return carry

    lax.fori_loop(0, n_chunks, reduce_chunk, 0, unroll=False)

    pltpu.sync_copy(out_v, out_hbm.at[pl.ds(base, b_per_w)])


def kernel(user, item, user_factors, item_factors):
    batch = user.shape[0]
    n_rows, n_factors = user_factors.shape
    assert n_factors == _F

    info = plsc.get_sparse_core_info()
    nw = info.num_cores * info.num_subcores
    b_per_w = batch // nw
    assert b_per_w * nw == batch and b_per_w % _L == 0

    mesh = plsc.VectorSubcoreMesh(core_axis_name="c", subcore_axis_name="s")

    mf = pl.kernel(
        functools.partial(_mf_body, b_per_w=b_per_w,
                          num_cores=info.num_cores),
        out_type=jax.ShapeDtypeStruct((batch,), jnp.float32),
        mesh=mesh,
        compiler_params=pltpu.CompilerParams(
            needs_layout_passes=False, use_tc_tiling_on_sc=False),
        scratch_types=[
            pltpu.VMEM((b_per_w,), jnp.int32),
            pltpu.VMEM((b_per_w,), jnp.int32),
            pltpu.VMEM((_F * b_per_w,), jnp.float32),
            pltpu.VMEM((_F * b_per_w,), jnp.float32),
            pltpu.VMEM((b_per_w,), jnp.float32),
            pltpu.SemaphoreType.DMA,
            pltpu.SemaphoreType.DMA,
        ],
    )
    return mf(user.astype(jnp.int32), item.astype(jnp.int32),
              user_factors.T, item_factors.T)


# speedup vs baseline: 5.7000x; 5.7000x over previous
import functools

import jax
import jax.numpy as jnp
from jax import lax
from jax.experimental import pallas as pl
from jax.experimental.pallas import tpu as pltpu
from jax.experimental.pallas import tpu_sc as plsc

_F = 32
_L = 16
_PASS = 256


def _mf_body(user_hbm, item_hbm, uf4_hbm, if4_hbm, out_hbm,
             uidx_v, iidx_v, u4_v, i4_v, urows_v, irows_v, out_v,
             usem, isem, *, b_per_w, num_cores):
    wid = lax.axis_index("s") * num_cores + lax.axis_index("c")
    base = wid * b_per_w

    pltpu.sync_copy(user_hbm.at[pl.ds(base, b_per_w)], uidx_v)
    pltpu.sync_copy(item_hbm.at[pl.ds(base, b_per_w)], iidx_v)

    n_chunks = b_per_w // _L

    def quarters(c, carry):
        uoff = c * _L
        u4_v[pl.ds(uoff, _L)] = uidx_v[pl.ds(uoff, _L)] >> 2
        i4_v[pl.ds(uoff, _L)] = iidx_v[pl.ds(uoff, _L)] >> 2
        return carry

    lax.fori_loop(0, n_chunks, quarters, 0, unroll=False)

    lane = lax.iota(jnp.int32, _L)

    for p in range(b_per_w // _PASS):
        pbase = p * _PASS
        ucp = pltpu.async_copy(
            uf4_hbm.at[u4_v.at[pl.ds(pbase, _PASS)]], urows_v, usem)
        icp = pltpu.async_copy(
            if4_hbm.at[i4_v.at[pl.ds(pbase, _PASS)]], irows_v, isem)
        ucp.wait()
        icp.wait()

        def group(g, carry, pbase=pbase):
            rbase = g * _L
            usub = (uidx_v[pl.ds(pbase + rbase, _L)] & 3) * _F
            vsub = (iidx_v[pl.ds(pbase + rbase, _L)] & 3) * _F
            acc = jnp.zeros((_L,), jnp.float32)
            for j in range(_L):
                r = rbase + j
                us = usub[j]
                vs = vsub[j]
                u0 = urows_v[r, pl.ds(us, _L)]
                u1 = urows_v[r, pl.ds(us + _L, _L)]
                v0 = irows_v[r, pl.ds(vs, _L)]
                v1 = irows_v[r, pl.ds(vs + _L, _L)]
                prod = u0 * v0 + u1 * v1
                acc = jnp.where(lane == j, jnp.sum(prod), acc)
            out_v[pl.ds(pbase + rbase, _L)] = acc
            return carry

        lax.fori_loop(0, _PASS // _L, group, 0, unroll=False)

    pltpu.sync_copy(out_v, out_hbm.at[pl.ds(base, b_per_w)])


def kernel(user, item, user_factors, item_factors):
    batch = user.shape[0]
    n_rows, n_factors = user_factors.shape
    assert n_factors == _F and n_rows % 4 == 0

    info = plsc.get_sparse_core_info()
    nw = info.num_cores * info.num_subcores
    b_per_w = batch // nw
    assert b_per_w * nw == batch and b_per_w % _PASS == 0

    mesh = plsc.VectorSubcoreMesh(core_axis_name="c", subcore_axis_name="s")

    mf = pl.kernel(
        functools.partial(_mf_body, b_per_w=b_per_w,
                          num_cores=info.num_cores),
        out_type=jax.ShapeDtypeStruct((batch,), jnp.float32),
        mesh=mesh,
        compiler_params=pltpu.CompilerParams(needs_layout_passes=False),
        scratch_types=[
            pltpu.VMEM((b_per_w,), jnp.int32),
            pltpu.VMEM((b_per_w,), jnp.int32),
            pltpu.VMEM((b_per_w,), jnp.int32),
            pltpu.VMEM((b_per_w,), jnp.int32),
            pltpu.VMEM((_PASS, 128), jnp.float32),
            pltpu.VMEM((_PASS, 128), jnp.float32),
            pltpu.VMEM((b_per_w,), jnp.float32),
            pltpu.SemaphoreType.DMA,
            pltpu.SemaphoreType.DMA,
        ],
    )
    return mf(user.astype(jnp.int32), item.astype(jnp.int32),
              user_factors.reshape(n_rows // 4, 4 * _F),
              item_factors.reshape(n_rows // 4, 4 * _F))


# speedup vs baseline: 5.7284x; 1.0050x over previous
import functools

import jax
import jax.numpy as jnp
from jax import lax
from jax.experimental import pallas as pl
from jax.experimental.pallas import tpu as pltpu
from jax.experimental.pallas import tpu_sc as plsc

_F = 32
_L = 16


def _mf_body(user_hbm, item_hbm, uf_hbm, if_hbm, out_hbm,
             uidx_v, iidx_v, urows_v, irows_v, out_v, usem, isem,
             *, b_per_w, num_cores):
    wid = lax.axis_index("s") * num_cores + lax.axis_index("c")
    base = wid * b_per_w

    pltpu.sync_copy(user_hbm.at[pl.ds(base, b_per_w)], uidx_v)
    pltpu.sync_copy(item_hbm.at[pl.ds(base, b_per_w)], iidx_v)

    ucp = pltpu.async_copy(uf_hbm.at[uidx_v], urows_v, usem)
    icp = pltpu.async_copy(if_hbm.at[iidx_v], irows_v, isem)
    ucp.wait()
    icp.wait()

    lane = lax.iota(jnp.int32, _L)

    def group(g, carry):
        rbase = g * _L
        acc = jnp.zeros((_L,), jnp.float32)
        for j in range(_L):
            r = rbase + j
            u0 = urows_v[r, pl.ds(0, _L)]
            u1 = urows_v[r, pl.ds(_L, _L)]
            v0 = irows_v[r, pl.ds(0, _L)]
            v1 = irows_v[r, pl.ds(_L, _L)]
            prod = u0 * v0 + u1 * v1
            acc = jnp.where(lane == j, jnp.sum(prod), acc)
        out_v[pl.ds(rbase, _L)] = acc
        return carry

    lax.fori_loop(0, b_per_w // _L, group, 0, unroll=False)

    pltpu.sync_copy(out_v, out_hbm.at[pl.ds(base, b_per_w)])


def kernel(user, item, user_factors, item_factors):
    batch = user.shape[0]
    n_factors = user_factors.shape[1]
    assert n_factors == _F

    info = plsc.get_sparse_core_info()
    nw = info.num_cores * info.num_subcores
    b_per_w = batch // nw
    assert b_per_w * nw == batch and b_per_w % _L == 0

    mesh = plsc.VectorSubcoreMesh(core_axis_name="c", subcore_axis_name="s")

    mf = pl.kernel(
        functools.partial(_mf_body, b_per_w=b_per_w, num_cores=info.num_cores),
        out_type=jax.ShapeDtypeStruct((batch,), jnp.float32),
        mesh=mesh,
        compiler_params=pltpu.CompilerParams(
            needs_layout_passes=False, use_tc_tiling_on_sc=False),
        scratch_types=[
            pltpu.VMEM((b_per_w,), jnp.int32),
            pltpu.VMEM((b_per_w,), jnp.int32),
            pltpu.VMEM((b_per_w, _F), jnp.float32),
            pltpu.VMEM((b_per_w, _F), jnp.float32),
            pltpu.VMEM((b_per_w,), jnp.float32),
            pltpu.SemaphoreType.DMA,
            pltpu.SemaphoreType.DMA,
        ],
    )
    return mf(user.astype(jnp.int32), item.astype(jnp.int32),
              user_factors, item_factors)


# speedup vs baseline: 5.7455x; 1.0030x over previous
import functools

import jax
import jax.numpy as jnp
from jax import lax
from jax.experimental import pallas as pl
from jax.experimental.pallas import tpu as pltpu
from jax.experimental.pallas import tpu_sc as plsc

_F = 32
_L = 16


def _mf_body(user_hbm, item_hbm, uf_hbm, if_hbm, out_hbm,
             uidx_v, iidx_v, urows_v, irows_v, out_v, usem, isem,
             *, b_per_w, num_cores):
    wid = lax.axis_index("s") * num_cores + lax.axis_index("c")
    base = wid * b_per_w

    pltpu.sync_copy(user_hbm.at[pl.ds(base, b_per_w)], uidx_v)
    pltpu.sync_copy(item_hbm.at[pl.ds(base, b_per_w)], iidx_v)

    n_ck = 8
    ck = b_per_w // n_ck
    copies = []
    for q in range(n_ck):
        sl = pl.ds(q * ck, ck)
        copies.append(pltpu.async_copy(
            uf_hbm.at[uidx_v.at[sl]], urows_v.at[sl], usem))
        copies.append(pltpu.async_copy(
            if_hbm.at[iidx_v.at[sl]], irows_v.at[sl], isem))
    for cp in copies:
        cp.wait()

    lane = lax.iota(jnp.int32, _L)

    def group(g, carry):
        rbase = g * _L
        acc = jnp.zeros((_L,), jnp.float32)
        for j in range(_L):
            r = rbase + j
            u0 = urows_v[r, pl.ds(0, _L)]
            u1 = urows_v[r, pl.ds(_L, _L)]
            v0 = irows_v[r, pl.ds(0, _L)]
            v1 = irows_v[r, pl.ds(_L, _L)]
            prod = u0 * v0 + u1 * v1
            acc = jnp.where(lane == j, jnp.sum(prod), acc)
        out_v[pl.ds(rbase, _L)] = acc
        return carry

    lax.fori_loop(0, b_per_w // _L, group, 0, unroll=False)

    pltpu.sync_copy(out_v, out_hbm.at[pl.ds(base, b_per_w)])


def kernel(user, item, user_factors, item_factors):
    batch = user.shape[0]
    n_factors = user_factors.shape[1]
    assert n_factors == _F

    info = plsc.get_sparse_core_info()
    nw = info.num_cores * info.num_subcores
    b_per_w = batch // nw
    assert b_per_w * nw == batch and b_per_w % _L == 0

    mesh = plsc.VectorSubcoreMesh(core_axis_name="c", subcore_axis_name="s")

    mf = pl.kernel(
        functools.partial(_mf_body, b_per_w=b_per_w, num_cores=info.num_cores),
        out_type=jax.ShapeDtypeStruct((batch,), jnp.float32),
        mesh=mesh,
        compiler_params=pltpu.CompilerParams(
            needs_layout_passes=False, use_tc_tiling_on_sc=False),
        scratch_types=[
            pltpu.VMEM((b_per_w,), jnp.int32),
            pltpu.VMEM((b_per_w,), jnp.int32),
            pltpu.VMEM((b_per_w, _F), jnp.float32),
            pltpu.VMEM((b_per_w, _F), jnp.float32),
            pltpu.VMEM((b_per_w,), jnp.float32),
            pltpu.SemaphoreType.DMA,
            pltpu.SemaphoreType.DMA,
        ],
    )
    return mf(user.astype(jnp.int32), item.astype(jnp.int32),
              user_factors, item_factors)
